# baseline (speedup 1.0000x reference)
import functools

import jax
import jax.numpy as jnp
from jax import lax
from jax.experimental import pallas as pl
from jax.experimental.pallas import tpu as pltpu
from jax.experimental.pallas import tpu_sc as plsc

_LAMDA = 0.01
_B = 16384
_D = 16
_N = 1_000_000

_NC, _NS = 2, 16
_NW = _NC * _NS
_BPW = _B // _NW
_CHUNK = 128
_NCHUNK = _BPW // _CHUNK
_NBLK = _CHUNK // _D

_NFB = _N // _CHUNK
_NTAIL = _N % _CHUNK
_TBASE = _NFB * _CHUNK
_PROWS = _NFB * 16 + 8
_CB_LO = _NFB // _NW
_CB_EXTRA = _NFB - _CB_LO * _NW


def _sc_detile(eu_t, ei_t):
    mesh = plsc.VectorSubcoreMesh(core_axis_name="c", subcore_axis_name="s")
    out_type = [jax.ShapeDtypeStruct((_PROWS, _CHUNK), jnp.float32)] * 2
    scratch = (
        [pltpu.VMEM((_D, _CHUNK), jnp.float32) for _ in range(4)]
        + [pltpu.SemaphoreType.DMA for _ in range(2)]
    )

    @functools.partial(
        pl.kernel, mesh=mesh, out_type=out_type, scratch_types=scratch,
        compiler_params=pltpu.CompilerParams(needs_layout_passes=False),
    )
    def body(eu, ei, pu, pi, vt0, vo0, vt1, vo1, sin, sout):
        wid = lax.axis_index("s") * _NC + lax.axis_index("c")
        start = wid * _CB_LO + jnp.minimum(wid, _CB_EXTRA)
        count = _CB_LO + jnp.where(wid < _CB_EXTRA, 1, 0)
        lane = lax.iota(jnp.int32, 16)

        for tbl, out, vt, vo in ((eu, pu, vt0, vo0), (ei, pi, vt1, vo1)):
            def colblock(i, _, tbl=tbl, out=out, vt=vt, vo=vo):
                c = start + i

                @pl.when(i < count)
                def _go():
                    pltpu.async_copy(
                        tbl.at[:, pl.ds(c * _CHUNK, _CHUNK)], vt, sin
                    ).wait()
                    for p in range(16):
                        for j in range(8):
                            col = jnp.full((16,), p * 8 + j, dtype=jnp.int32)
                            vo[p, pl.ds(j * _D, _D)] = plsc.load_gather(
                                vt, [lane, col]
                            )
                    pltpu.async_copy(
                        vo, out.at[pl.ds(c * _D, _D)], sout
                    ).wait()

                return _

            lax.fori_loop(0, _CB_LO + 1, colblock, None)

    return body(eu_t, ei_t)


def _sc_gather_reduce(pu, pi, tail_u, tail_i, idx2d):
    mesh = plsc.VectorSubcoreMesh(core_axis_name="c", subcore_axis_name="s")
    out_type = [jax.ShapeDtypeStruct((_B,), jnp.float32) for _ in range(7)]
    scratch = (
        [pltpu.VMEM((_NCHUNK, _CHUNK), jnp.int32) for _ in range(6)]
        + [pltpu.VMEM((_NCHUNK, _CHUNK), jnp.int32) for _ in range(6)]
        + [pltpu.VMEM((_CHUNK, _CHUNK), jnp.float32) for _ in range(6)]
        + [pltpu.VMEM((_BPW,), jnp.float32) for _ in range(7)]
        + [pltpu.VMEM((_NTAIL, _D), jnp.float32) for _ in range(2)]
        + [pltpu.SemaphoreType.DMA for _ in range(6)]
    )

    @functools.partial(
        pl.kernel, mesh=mesh, out_type=out_type, scratch_types=scratch,
        compiler_params=pltpu.CompilerParams(
            use_tc_tiling_on_sc=False, needs_layout_passes=False
        ),
    )
    def body(pu_h, pi_h, tu_h, ti_h, idx_hbm,
             xo, q0, q1, q2, q3, q4, q5,
             iv0, iv1, iv2, iv3, iv4, iv5,
             rw0, rw1, rw2, rw3, rw4, rw5,
             rv0, rv1, rv2, rv3, rv4, rv5,
             xv, w0, w1, w2, w3, w4, w5,
             tu_v, ti_v,
             s0, s1, s2, s3, s4, s5):
        ivs = (iv0, iv1, iv2, iv3, iv4, iv5)
        rws = (rw0, rw1, rw2, rw3, rw4, rw5)
        rvs = (rv0, rv1, rv2, rv3, rv4, rv5)
        wvs = (w0, w1, w2, w3, w4, w5)
        qouts = (q0, q1, q2, q3, q4, q5)
        sems = (s0, s1, s2, s3, s4, s5)
        tables = (pu_h, pi_h, pu_h, pi_h, pu_h, pi_h)

        tails = (tu_v, ti_v, tu_v, ti_v, tu_v, ti_v)
        wid = lax.axis_index("s") * _NC + lax.axis_index("c")
        crow = wid * _NCHUNK
        lane = lax.iota(jnp.int32, 16)

        pltpu.sync_copy(tu_h, tu_v)
        pltpu.sync_copy(ti_h, ti_v)

        for t in range(6):
            pltpu.sync_copy(idx_hbm.at[t, pl.ds(crow, _NCHUNK)], ivs[t])
        for t in range(6):
            for j in range(_NCHUNK):
                for g in range(_CHUNK // _D):
                    v = ivs[t][j, pl.ds(g * _D, _D)]
                    rws[t][j, pl.ds(g * _D, _D)] = lax.shift_right_logical(
                        v, 3
                    )

        for j in range(_NCHUNK):
            handles = []
            for t in range(6):
                handles.append(
                    pltpu.async_copy(
                        tables[t].at[rws[t].at[j]], rvs[t], sems[t]
                    )
                )
            for h in handles:
                h.wait()
            for m in range(_NBLK):
                row = m * _D + lane
                idvecs = [ivs[t][j, pl.ds(m * _D, _D)] for t in range(6)]
                offs = [
                    lax.shift_left(
                        jnp.bitwise_and(idvecs[t], jnp.int32(7)), 4
                    )
                    for t in range(6)
                ]
                tmask = [idvecs[t] >= _TBASE for t in range(6)]
                tidx = [
                    jnp.maximum(idvecs[t] - _TBASE, 0) for t in range(6)
                ]
                zero = jnp.zeros((16,), jnp.float32)
                x = zero
                qs = [zero] * 6
                for d in range(_D):
                    dcol = jnp.full((16,), d, dtype=jnp.int32)
                    c = [
                        jnp.where(
                            tmask[t],
                            plsc.load_gather(tails[t], [tidx[t], dcol]),
                            plsc.load_gather(rvs[t], [row, offs[t] + d]),
                        )
                        for t in range(6)
                    ]
                    g = c[0] + c[1]
                    gp = c[2] + c[3]
                    gn = c[4] + c[5]
                    x = x + g * (gp - gn)
                    for t in range(6):
                        qs[t] = qs[t] + c[t] * c[t]
                s = pl.ds(j * _CHUNK + m * _D, _D)
                xv[s] = x
                for t in range(6):
                    wvs[t][s] = qs[t]

        base = wid * _BPW
        pltpu.sync_copy(xv, xo.at[pl.ds(base, _BPW)])
        for t in range(6):
            pltpu.sync_copy(wvs[t], qouts[t].at[pl.ds(base, _BPW)])

    return body(pu, pi, tail_u, tail_i, idx2d)


def _tc_reduce(x, q0, q1, q2, q3, q4, q5):
    def body(x_r, q0_r, q1_r, q2_r, q3_r, q4_r, q5_r, loss_r, reg_r):
        x = x_r[...]
        loss_r[0, 0] = jnp.sum(jnp.log1p(jnp.exp(-x)))
        reg = 0.0
        for q in (q0_r, q1_r, q2_r, q3_r, q4_r, q5_r):
            reg = reg + jnp.sum(jnp.sqrt(q[...]))
        reg_r[0, 0] = reg * _LAMDA

    loss, reg = pl.pallas_call(
        body,
        out_shape=[jax.ShapeDtypeStruct((1, 1), jnp.float32)] * 2,
        in_specs=[pl.BlockSpec(memory_space=pltpu.VMEM)] * 7,
        out_specs=[pl.BlockSpec(memory_space=pltpu.SMEM)] * 2,
    )(x, q0, q1, q2, q3, q4, q5)
    return loss[0, 0], reg[0, 0]


def kernel(embed_user, embed_item, user, item, pos_user, pos_item, neg_user, neg_item):
    idx2d = jnp.stack(
        [user, item, pos_user, pos_item, neg_user, neg_item]
    ).reshape(6, _B // _CHUNK, _CHUNK)
    pu, pi = _sc_detile(embed_user.T, embed_item.T)
    x, q0, q1, q2, q3, q4, q5 = _sc_gather_reduce(
        pu, pi, embed_user[_TBASE:], embed_item[_TBASE:], idx2d
    )
    sq = _B // _CHUNK
    return _tc_reduce(
        x.reshape(sq, _CHUNK),
        q0.reshape(sq, _CHUNK),
        q1.reshape(sq, _CHUNK),
        q2.reshape(sq, _CHUNK),
        q3.reshape(sq, _CHUNK),
        q4.reshape(sq, _CHUNK),
        q5.reshape(sq, _CHUNK),
    )
